# baseline (device time: 315720 ns/iter reference)
import numpy as np
import jax
import jax.numpy as jnp
from jax import lax
from jax.experimental import pallas as pl
from jax.experimental.pallas import tpu as pltpu

N_DEV = 32
B_LOC = 2
SQ = 128
D = 512
H_LOC = 4
DH = 64
ROWS = B_LOC * SQ

WIRE_DTYPE = jnp.bfloat16

_sem_signal = getattr(pltpu, "semaphore_signal", None) or getattr(pl, "semaphore_signal")
_sem_wait = getattr(pltpu, "semaphore_wait", None) or getattr(pl, "semaphore_wait")
_CompilerParams = getattr(pltpu, "CompilerParams", None) or getattr(
    pltpu, "TPUCompilerParams"
)


def _rope_tables():
    inv = 1.0 / (10000.0 ** (np.arange(0, DH, 2) / DH))
    pos = np.arange(SQ)[:, None] * inv[None, :]
    cos = np.repeat(np.cos(pos), 2, axis=-1).astype(np.float32)
    sin = np.repeat(np.sin(pos), 2, axis=-1).astype(np.float32)
    R = np.zeros((DH, DH), np.float32)
    k = np.arange(0, DH, 2)
    R[k + 1, k] = -1.0
    R[k, k + 1] = 1.0
    return cos, sin, R


def kernel(x, Wq, Wk, Wv, Wo):
    cos_np, sin_np, R_np = _rope_tables()
    cos_in = jnp.asarray(cos_np)
    sin_in = jnp.asarray(sin_np)
    R_in = jnp.asarray(R_np)

    def body(x_ref, wq_ref, wk_ref, wv_ref, wo_ref, cos_ref, sin_ref, r_ref,
             out_ref, xbuf, abuf, x_ssem, x_rsem, a_ssem, a_rsem,
             x_credit, a_credit):
        my = lax.axis_index("i")
        left = lax.rem(my - 1 + N_DEV, N_DEV)
        right = lax.rem(my + 1, N_DEV)

        barrier_sem = pltpu.get_barrier_semaphore()
        for nbr in (left, right):
            _sem_signal(barrier_sem, inc=1, device_id=(nbr,),
                        device_id_type=pl.DeviceIdType.MESH)
        _sem_wait(barrier_sem, 2)

        wq = wq_ref[...].astype(WIRE_DTYPE)
        wk = wk_ref[...].astype(WIRE_DTYPE)
        wv = wv_ref[...].astype(WIRE_DTYPE)
        wo = wo_ref[...].astype(WIRE_DTYPE)
        cos = cos_ref[...]
        sin = sin_ref[...]
        R = r_ref[...]

        def partial_for(xc):
            q = jnp.dot(xc, wq, preferred_element_type=jnp.float32)
            k = jnp.dot(xc, wk, preferred_element_type=jnp.float32)
            v = jnp.dot(xc, wv, preferred_element_type=jnp.float32)
            parts = []
            for b in range(B_LOC):
                pb = jnp.zeros((SQ, D), jnp.float32)
                for h in range(H_LOC):
                    rs = slice(b * SQ, (b + 1) * SQ)
                    cs = slice(h * DH, (h + 1) * DH)
                    qb = q[rs, cs]
                    kb = k[rs, cs]
                    vb = v[rs, cs].astype(WIRE_DTYPE)
                    qr = (qb * cos + jnp.dot(qb, R, preferred_element_type=jnp.float32) * sin).astype(WIRE_DTYPE)
                    kr = (kb * cos + jnp.dot(kb, R, preferred_element_type=jnp.float32) * sin).astype(WIRE_DTYPE)
                    sc = lax.dot_general(
                        qr, kr, (((1,), (1,)), ((), ())),
                        preferred_element_type=jnp.float32,
                    ) * 0.125
                    m = jnp.max(sc, axis=1, keepdims=True)
                    e = jnp.exp(sc - m)
                    w = (e / jnp.sum(e, axis=1, keepdims=True)).astype(WIRE_DTYPE)
                    ctx = jnp.dot(w, vb, preferred_element_type=jnp.float32)
                    pb = pb + jnp.dot(ctx.astype(WIRE_DTYPE), wo[h * DH:(h + 1) * DH, :],
                                      preferred_element_type=jnp.float32)
                parts.append(pb)
            return jnp.concatenate(parts, axis=0)

        def mk(buf, s_, r_, ssem, rsem):
            return pltpu.make_async_remote_copy(
                src_ref=buf.at[s_],
                dst_ref=buf.at[r_],
                send_sem=ssem.at[s_],
                recv_sem=rsem.at[r_],
                device_id=(right,),
                device_id_type=pl.DeviceIdType.MESH,
            )

        def signal(sem, nbr):
            _sem_signal(sem, inc=1, device_id=(nbr,),
                        device_id_type=pl.DeviceIdType.MESH)

        xc0 = x_ref[...].reshape(ROWS, D)
        xbuf[0] = xc0.astype(WIRE_DTYPE)
        x_send0 = mk(xbuf, 0, 1, x_ssem, x_rsem)
        x_send0.start()
        abuf[0] = partial_for(xc0.astype(WIRE_DTYPE)).astype(WIRE_DTYPE)
        a_send0 = mk(abuf, 0, 1, a_ssem, a_rsem)
        a_send0.start()
        x_send0.wait_send()
        signal(x_credit, left)
        a_send0.wait_send()
        signal(a_credit, left)

        def step(t, carry):
            s = lax.rem(t, 2)
            r = 1 - s
            x_recv = mk(xbuf, r, s, x_ssem, x_rsem)
            x_recv.wait_recv()
            x_send = mk(xbuf, s, r, x_ssem, x_rsem)

            @pl.when(t < N_DEV - 1)
            def _():
                _sem_wait(x_credit, 1)
                x_send.start()

            p = partial_for(xbuf[s])

            a_recv = mk(abuf, r, s, a_ssem, a_rsem)
            a_recv.wait_recv()
            abuf[s] = (abuf[s].astype(jnp.float32) + p).astype(WIRE_DTYPE)
            _sem_wait(a_credit, 1)
            a_send = mk(abuf, s, r, a_ssem, a_rsem)
            a_send.start()

            @pl.when(t < N_DEV - 1)
            def _():
                x_send.wait_send()

            @pl.when(t < N_DEV - 2)
            def _():
                signal(x_credit, left)

            a_send.wait_send()

            @pl.when(t < N_DEV - 1)
            def _():
                signal(a_credit, left)

            return carry

        lax.fori_loop(1, N_DEV, step, 0)

        a_final = mk(abuf, 1, 0, a_ssem, a_rsem)
        a_final.wait_recv()
        out_ref[...] = abuf[0].astype(jnp.float32).reshape(B_LOC, SQ, D)

    out_shape = jax.ShapeDtypeStruct((B_LOC, SQ, D), jnp.float32)
    vmem = pl.BlockSpec(memory_space=pltpu.VMEM)
    return pl.pallas_call(
        body,
        out_shape=out_shape,
        in_specs=[vmem] * 8,
        out_specs=vmem,
        scratch_shapes=[
            pltpu.VMEM((2, ROWS, D), WIRE_DTYPE),
            pltpu.VMEM((2, ROWS, D), WIRE_DTYPE),
            pltpu.SemaphoreType.DMA((2,)),
            pltpu.SemaphoreType.DMA((2,)),
            pltpu.SemaphoreType.DMA((2,)),
            pltpu.SemaphoreType.DMA((2,)),
            pltpu.SemaphoreType.REGULAR,
            pltpu.SemaphoreType.REGULAR,
        ],
        compiler_params=_CompilerParams(collective_id=0),
    )(x, Wq, Wk, Wv, Wo, cos_in, sin_in, R_in)


# device time: 309677 ns/iter; 1.0195x vs baseline; 1.0195x over previous
import numpy as np
import jax
import jax.numpy as jnp
from jax import lax
from jax.experimental import pallas as pl
from jax.experimental.pallas import tpu as pltpu

N_DEV = 32
B_LOC = 2
SQ = 128
D = 512
H_LOC = 4
DH = 64

WIRE_DTYPE = jnp.bfloat16

_sem_signal = getattr(pltpu, "semaphore_signal", None) or getattr(pl, "semaphore_signal")
_sem_wait = getattr(pltpu, "semaphore_wait", None) or getattr(pl, "semaphore_wait")
_CompilerParams = getattr(pltpu, "CompilerParams", None) or getattr(
    pltpu, "TPUCompilerParams"
)


def _rope_tables():
    inv = 1.0 / (10000.0 ** (np.arange(0, DH, 2) / DH))
    pos = np.arange(SQ)[:, None] * inv[None, :]
    cos = np.repeat(np.cos(pos), 2, axis=-1).astype(np.float32)
    sin = np.repeat(np.sin(pos), 2, axis=-1).astype(np.float32)
    R = np.zeros((DH, DH), np.float32)
    k = np.arange(0, DH, 2)
    R[k + 1, k] = -1.0
    R[k, k + 1] = 1.0
    return cos, sin, R


def kernel(x, Wq, Wk, Wv, Wo):
    cos_np, sin_np, R_np = _rope_tables()
    cos_in = jnp.asarray(cos_np)
    sin_in = jnp.asarray(sin_np)
    R_in = jnp.asarray(R_np)

    def body(x_ref, wq_ref, wk_ref, wv_ref, wo_ref, cos_ref, sin_ref, r_ref,
             out_ref,
             xbr, abr, xbl, abl,
             xr_ssem, xr_rsem, ar_ssem, ar_rsem,
             xl_ssem, xl_rsem, al_ssem, al_rsem,
             xr_credit, ar_credit, xl_credit, al_credit):
        my = lax.axis_index("i")
        left = lax.rem(my - 1 + N_DEV, N_DEV)
        right = lax.rem(my + 1, N_DEV)

        barrier_sem = pltpu.get_barrier_semaphore()
        for nbr in (left, right):
            _sem_signal(barrier_sem, inc=1, device_id=(nbr,),
                        device_id_type=pl.DeviceIdType.MESH)
        _sem_wait(barrier_sem, 2)

        wq = wq_ref[...].astype(WIRE_DTYPE)
        wk = wk_ref[...].astype(WIRE_DTYPE)
        wv = wv_ref[...].astype(WIRE_DTYPE)
        wo = wo_ref[...].astype(WIRE_DTYPE)
        cos = cos_ref[...]
        sin = sin_ref[...]
        R = r_ref[...]

        def partial_half(xc):
            q = jnp.dot(xc, wq, preferred_element_type=jnp.float32)
            k = jnp.dot(xc, wk, preferred_element_type=jnp.float32)
            v = jnp.dot(xc, wv, preferred_element_type=jnp.float32)
            pb = jnp.zeros((SQ, D), jnp.float32)
            for h in range(H_LOC):
                cs = slice(h * DH, (h + 1) * DH)
                qb = q[:, cs]
                kb = k[:, cs]
                vb = v[:, cs].astype(WIRE_DTYPE)
                qr = (qb * cos + jnp.dot(qb, R, preferred_element_type=jnp.float32) * sin).astype(WIRE_DTYPE)
                kr = (kb * cos + jnp.dot(kb, R, preferred_element_type=jnp.float32) * sin).astype(WIRE_DTYPE)
                sc = lax.dot_general(
                    qr, kr, (((1,), (1,)), ((), ())),
                    preferred_element_type=jnp.float32,
                ) * 0.125
                m = jnp.max(sc, axis=1, keepdims=True)
                e = jnp.exp(sc - m)
                w = (e / jnp.sum(e, axis=1, keepdims=True)).astype(WIRE_DTYPE)
                ctx = jnp.dot(w, vb, preferred_element_type=jnp.float32)
                pb = pb + jnp.dot(ctx.astype(WIRE_DTYPE), wo[h * DH:(h + 1) * DH, :],
                                  preferred_element_type=jnp.float32)
            return pb

        def mk(buf, s_, r_, ssem, rsem, nbr):
            return pltpu.make_async_remote_copy(
                src_ref=buf.at[s_],
                dst_ref=buf.at[r_],
                send_sem=ssem.at[s_],
                recv_sem=rsem.at[r_],
                device_id=(nbr,),
                device_id_type=pl.DeviceIdType.MESH,
            )

        def signal(sem, nbr):
            _sem_signal(sem, inc=1, device_id=(nbr,),
                        device_id_type=pl.DeviceIdType.MESH)

        xbr[0] = x_ref[0].astype(WIRE_DTYPE)
        xr_send0 = mk(xbr, 0, 1, xr_ssem, xr_rsem, right)
        xr_send0.start()
        xbl[0] = x_ref[1].astype(WIRE_DTYPE)
        xl_send0 = mk(xbl, 0, 1, xl_ssem, xl_rsem, left)
        xl_send0.start()
        abr[0] = partial_half(xbr[0]).astype(WIRE_DTYPE)
        ar_send0 = mk(abr, 0, 1, ar_ssem, ar_rsem, right)
        ar_send0.start()
        abl[0] = partial_half(xbl[0]).astype(WIRE_DTYPE)
        al_send0 = mk(abl, 0, 1, al_ssem, al_rsem, left)
        al_send0.start()
        xr_send0.wait_send()
        signal(xr_credit, left)
        xl_send0.wait_send()
        signal(xl_credit, right)
        ar_send0.wait_send()
        signal(ar_credit, left)
        al_send0.wait_send()
        signal(al_credit, right)

        def step(t, carry):
            s = lax.rem(t, 2)
            r = 1 - s

            xr_recv = mk(xbr, r, s, xr_ssem, xr_rsem, right)
            xr_recv.wait_recv()
            xr_send = mk(xbr, s, r, xr_ssem, xr_rsem, right)

            @pl.when(t < N_DEV - 1)
            def _():
                _sem_wait(xr_credit, 1)
                xr_send.start()

            xl_recv = mk(xbl, r, s, xl_ssem, xl_rsem, left)
            xl_recv.wait_recv()
            xl_send = mk(xbl, s, r, xl_ssem, xl_rsem, left)

            @pl.when(t < N_DEV - 1)
            def _():
                _sem_wait(xl_credit, 1)
                xl_send.start()

            pr = partial_half(xbr[s])
            pl_ = partial_half(xbl[s])

            ar_recv = mk(abr, r, s, ar_ssem, ar_rsem, right)
            ar_recv.wait_recv()
            abr[s] = (abr[s].astype(jnp.float32) + pr).astype(WIRE_DTYPE)
            _sem_wait(ar_credit, 1)
            ar_send = mk(abr, s, r, ar_ssem, ar_rsem, right)
            ar_send.start()

            al_recv = mk(abl, r, s, al_ssem, al_rsem, left)
            al_recv.wait_recv()
            abl[s] = (abl[s].astype(jnp.float32) + pl_).astype(WIRE_DTYPE)
            _sem_wait(al_credit, 1)
            al_send = mk(abl, s, r, al_ssem, al_rsem, left)
            al_send.start()

            @pl.when(t < N_DEV - 1)
            def _():
                xr_send.wait_send()
                xl_send.wait_send()

            @pl.when(t < N_DEV - 2)
            def _():
                signal(xr_credit, left)
                signal(xl_credit, right)

            ar_send.wait_send()
            al_send.wait_send()

            @pl.when(t < N_DEV - 1)
            def _():
                signal(ar_credit, left)
                signal(al_credit, right)

            return carry

        lax.fori_loop(1, N_DEV, step, 0)

        ar_final = mk(abr, 1, 0, ar_ssem, ar_rsem, right)
        ar_final.wait_recv()
        out_ref[0] = abr[0].astype(jnp.float32)
        al_final = mk(abl, 1, 0, al_ssem, al_rsem, left)
        al_final.wait_recv()
        out_ref[1] = abl[0].astype(jnp.float32)

    out_shape = jax.ShapeDtypeStruct((B_LOC, SQ, D), jnp.float32)
    vmem = pl.BlockSpec(memory_space=pltpu.VMEM)
    dma2 = pltpu.SemaphoreType.DMA((2,))
    return pl.pallas_call(
        body,
        out_shape=out_shape,
        in_specs=[vmem] * 8,
        out_specs=vmem,
        scratch_shapes=[
            pltpu.VMEM((2, SQ, D), WIRE_DTYPE),
            pltpu.VMEM((2, SQ, D), WIRE_DTYPE),
            pltpu.VMEM((2, SQ, D), WIRE_DTYPE),
            pltpu.VMEM((2, SQ, D), WIRE_DTYPE),
            dma2, dma2, dma2, dma2,
            dma2, dma2, dma2, dma2,
            pltpu.SemaphoreType.REGULAR,
            pltpu.SemaphoreType.REGULAR,
            pltpu.SemaphoreType.REGULAR,
            pltpu.SemaphoreType.REGULAR,
        ],
        compiler_params=_CompilerParams(collective_id=0),
    )(x, Wq, Wk, Wv, Wo, cos_in, sin_in, R_in)
